# baseline (speedup 1.0000x reference)
import jax
import jax.numpy as jnp
from jax.experimental import pallas as pl
from jax.experimental.pallas import tpu as pltpu

_BLOCK_ROWS = 1024

_DN = (((1,), (1,)), ((), ()))


def _fused_linear_kernel(x_ref, wc_ref, bc_ref, wb_ref, bb_ref,
                         scores_t_ref, deltas_t_ref):
    x = x_ref[...]
    scores_t_ref[...] = (
        jax.lax.dot_general(wc_ref[...], x, _DN,
                            preferred_element_type=jnp.float32)
        + bc_ref[...]
    )
    deltas_t_ref[...] = (
        jax.lax.dot_general(wb_ref[...], x, _DN,
                            preferred_element_type=jnp.float32)
        + bb_ref[...]
    )


@jax.jit
def kernel(x, W_cls, b_cls, W_box, b_box):
    if x.ndim > 2:
        x = x.reshape(x.shape[0], -1)
    n, in_dim = x.shape
    n_cls = W_cls.shape[0]
    n_box = W_box.shape[0]

    bc = b_cls.reshape(n_cls, 1)
    bb = b_box.reshape(n_box, 1)

    grid = (pl.cdiv(n, _BLOCK_ROWS),)
    scores_t, deltas_t = pl.pallas_call(
        _fused_linear_kernel,
        grid=grid,
        in_specs=[
            pl.BlockSpec((_BLOCK_ROWS, in_dim), lambda i: (i, 0)),
            pl.BlockSpec((n_cls, in_dim), lambda i: (0, 0)),
            pl.BlockSpec((n_cls, 1), lambda i: (0, 0)),
            pl.BlockSpec((n_box, in_dim), lambda i: (0, 0)),
            pl.BlockSpec((n_box, 1), lambda i: (0, 0)),
        ],
        out_specs=[
            pl.BlockSpec((n_cls, _BLOCK_ROWS), lambda i: (0, i)),
            pl.BlockSpec((n_box, _BLOCK_ROWS), lambda i: (0, i)),
        ],
        out_shape=[
            jax.ShapeDtypeStruct((n_cls, n), jnp.float32),
            jax.ShapeDtypeStruct((n_box, n), jnp.float32),
        ],
        compiler_params=pltpu.CompilerParams(
            dimension_semantics=("arbitrary",),
        ),
    )(x, W_cls, bc, W_box, bb)
    return (scores_t.T, deltas_t.T)


# speedup vs baseline: 1.1443x; 1.1443x over previous
import jax
import jax.numpy as jnp
from jax.experimental import pallas as pl
from jax.experimental.pallas import tpu as pltpu

_BLOCK_ROWS = 2560

_DN = (((1,), (1,)), ((), ()))


def _fused_linear_kernel(x_ref, wc_ref, bc_ref, wb_ref, bb_ref,
                         scores_t_ref, deltas_t_ref):
    x = x_ref[...]
    scores_t_ref[...] = (
        jax.lax.dot_general(wc_ref[...], x, _DN,
                            preferred_element_type=jnp.float32)
        + bc_ref[...]
    )
    deltas_t_ref[...] = (
        jax.lax.dot_general(wb_ref[...], x, _DN,
                            preferred_element_type=jnp.float32)
        + bb_ref[...]
    )


@jax.jit
def kernel(x, W_cls, b_cls, W_box, b_box):
    if x.ndim > 2:
        x = x.reshape(x.shape[0], -1)
    n, in_dim = x.shape
    n_cls = W_cls.shape[0]
    n_box = W_box.shape[0]

    bc = b_cls.reshape(n_cls, 1)
    bb = b_box.reshape(n_box, 1)

    grid = (pl.cdiv(n, _BLOCK_ROWS),)
    scores_t, deltas_t = pl.pallas_call(
        _fused_linear_kernel,
        grid=grid,
        in_specs=[
            pl.BlockSpec((_BLOCK_ROWS, in_dim), lambda i: (i, 0)),
            pl.BlockSpec((n_cls, in_dim), lambda i: (0, 0)),
            pl.BlockSpec((n_cls, 1), lambda i: (0, 0)),
            pl.BlockSpec((n_box, in_dim), lambda i: (0, 0)),
            pl.BlockSpec((n_box, 1), lambda i: (0, 0)),
        ],
        out_specs=[
            pl.BlockSpec((n_cls, _BLOCK_ROWS), lambda i: (0, i)),
            pl.BlockSpec((n_box, _BLOCK_ROWS), lambda i: (0, i)),
        ],
        out_shape=[
            jax.ShapeDtypeStruct((n_cls, n), jnp.float32),
            jax.ShapeDtypeStruct((n_box, n), jnp.float32),
        ],
        compiler_params=pltpu.CompilerParams(
            dimension_semantics=("arbitrary",),
        ),
    )(x, W_cls, bc, W_box, bb)
    return (scores_t.T, deltas_t.T)


# speedup vs baseline: 1.1655x; 1.0186x over previous
import jax
import jax.numpy as jnp
from jax.experimental import pallas as pl
from jax.experimental.pallas import tpu as pltpu

_BLOCK_ROWS = 2560

_DN = (((1,), (1,)), ((), ()))


def _fused_linear_kernel(x_ref, wc_ref, bc_ref, wb_ref, bb_ref,
                         scores_t_ref, deltas_t_ref):
    x = x_ref[...]
    scores_t_ref[...] = (
        jax.lax.dot_general(wc_ref[...], x, _DN,
                            preferred_element_type=jnp.float32)
        + bc_ref[...].T
    )
    deltas_t_ref[...] = (
        jax.lax.dot_general(wb_ref[...], x, _DN,
                            preferred_element_type=jnp.float32)
        + bb_ref[...].T
    )


@jax.jit
def kernel(x, W_cls, b_cls, W_box, b_box):
    if x.ndim > 2:
        x = x.reshape(x.shape[0], -1)
    n, in_dim = x.shape
    n_cls = W_cls.shape[0]
    n_box = W_box.shape[0]

    bc = b_cls.reshape(1, n_cls)
    bb = b_box.reshape(1, n_box)

    grid = (pl.cdiv(n, _BLOCK_ROWS),)
    scores_t, deltas_t = pl.pallas_call(
        _fused_linear_kernel,
        grid=grid,
        in_specs=[
            pl.BlockSpec((_BLOCK_ROWS, in_dim), lambda i: (i, 0)),
            pl.BlockSpec((n_cls, in_dim), lambda i: (0, 0)),
            pl.BlockSpec((1, n_cls), lambda i: (0, 0)),
            pl.BlockSpec((n_box, in_dim), lambda i: (0, 0)),
            pl.BlockSpec((1, n_box), lambda i: (0, 0)),
        ],
        out_specs=[
            pl.BlockSpec((n_cls, _BLOCK_ROWS), lambda i: (0, i)),
            pl.BlockSpec((n_box, _BLOCK_ROWS), lambda i: (0, i)),
        ],
        out_shape=[
            jax.ShapeDtypeStruct((n_cls, n), jnp.float32),
            jax.ShapeDtypeStruct((n_box, n), jnp.float32),
        ],
        compiler_params=pltpu.CompilerParams(
            dimension_semantics=("arbitrary",),
        ),
    )(x, W_cls, bc, W_box, bb)
    return (scores_t.T, deltas_t.T)
